# Initial kernel scaffold; baseline (speedup 1.0000x reference)
import jax
import jax.numpy as jnp
from jax.experimental import pallas as pl


def kernel(x, W_in, b_in, ln_scale, ln_bias, jump_destinations, W_gate, b_gate, context_strength, W_out, b_out, pointer_init, holonomy_init):
    raise NotImplementedError("write your pallas kernel here")


# speedup vs baseline: 1.3598x; 1.3598x over previous
import jax
import jax.numpy as jnp
from jax.experimental import pallas as pl
from jax.experimental.pallas import tpu as pltpu

B, T, IN = 1024, 32, 32
M, D, NOUT = 256, 64, 128
K = 2
NTAP = 2 * K + 1
TEMP = 8.0
EPS = 1e-5

BC = 128
GRID = B // BC


def _recurrence_kernel(x_ref, w_in_ref, b_in_ref, ln_s_ref, ln_b_ref,
                       jd_ref, w_gate_ref, b_gate_ref, cscale_ref,
                       w_out_ref, b_out_ref, ptr_ref, out_ref,
                       su_hist, base_hist, w_hist):
    ptr = ptr_ref[0, 0, :]
    hidden = jnp.zeros((BC, D), jnp.float32)
    su_hist[...] = jnp.zeros((T, BC, D), jnp.float32)
    cscale = cscale_ref[0, 0]
    s_iota = jax.lax.broadcasted_iota(jnp.int32, (T, BC), 0)

    for t in range(T):
        e = jnp.tanh(
            jnp.dot(x_ref[:, t, :], w_in_ref[...],
                    preferred_element_type=jnp.float32) + b_in_ref[0, :])
        base = jnp.floor(ptr).astype(jnp.int32)
        base = jnp.clip(base, 0, M - 1)
        att = []
        for i in range(NTAP):
            idx_f = jnp.mod(base + (i - K), M).astype(jnp.float32)
            delta = jnp.remainder(idx_f - ptr + M / 2.0, float(M)) - M / 2.0
            att.append(-(delta * delta) / TEMP)
        mx = jnp.maximum(jnp.maximum(jnp.maximum(att[0], att[1]),
                                     jnp.maximum(att[2], att[3])), att[4])
        ex = [jnp.exp(a - mx) for a in att]
        tot = ((ex[0] + ex[1]) + (ex[2] + ex[3])) + ex[4]
        w = [ei / tot for ei in ex]

        bh = base_hist[...]
        db = jnp.mod(bh - base[None, :], M)
        dd = jnp.where(db > M // 2, db - M, db)
        c = jnp.zeros((T, BC), jnp.float32)
        for d in range(-2 * K, 2 * K + 1):
            lo, hi = max(0, -d), min(NTAP - 1, NTAP - 1 - d)
            corr = jnp.zeros((T, BC), jnp.float32)
            for j in range(lo, hi + 1):
                corr = corr + w[j + d][None, :] * w_hist[j]
            c = jnp.where(dd == d, corr, c)
        c = jnp.where(s_iota < t, c, 0.0)
        context = jnp.sum(c[:, :, None] * su_hist[...], axis=0)

        combined = e + cscale * context
        su = jnp.tanh(combined + hidden)
        mu = jnp.mean(su, axis=-1, keepdims=True)
        var = jnp.mean((su - mu) ** 2, axis=-1, keepdims=True)
        hidden = (su - mu) / jnp.sqrt(var + EPS) * ln_s_ref[0, :] + ln_b_ref[0, :]

        su_hist[t] = su
        base_hist[t] = base
        for j in range(NTAP):
            w_hist[j, t] = w[j]

        pos = jnp.clip(ptr.astype(jnp.int32), 0, M - 1)
        jt = jnp.sum(
            jnp.where(jax.lax.broadcasted_iota(jnp.int32, (BC, M), 1) == pos[:, None],
                      jd_ref[0, :][None, :], 0.0), axis=1)
        jl = (jnp.dot(su, w_gate_ref[...],
                      preferred_element_type=jnp.float32)[:, 0] + b_gate_ref[0, 0])
        walk = jnp.remainder(ptr + 1.0, float(M))
        ptr = jnp.where(jl > 0.0, jt, walk)

    out_ref[...] = (jnp.dot(hidden, w_out_ref[...],
                            preferred_element_type=jnp.float32) + b_out_ref[0, :])


def kernel(x, W_in, b_in, ln_scale, ln_bias, jump_destinations, W_gate,
           b_gate, context_strength, W_out, b_out, pointer_init, holonomy_init):
    cscale = jax.nn.sigmoid(context_strength).reshape(1, 1)
    logits = pl.pallas_call(
        _recurrence_kernel,
        grid=(GRID,),
        in_specs=[
            pl.BlockSpec((BC, T, IN), lambda i: (i, 0, 0)),
            pl.BlockSpec((IN, D), lambda i: (0, 0)),
            pl.BlockSpec((1, D), lambda i: (0, 0)),
            pl.BlockSpec((1, D), lambda i: (0, 0)),
            pl.BlockSpec((1, D), lambda i: (0, 0)),
            pl.BlockSpec((1, M), lambda i: (0, 0)),
            pl.BlockSpec((D, 1), lambda i: (0, 0)),
            pl.BlockSpec((1, 1), lambda i: (0, 0)),
            pl.BlockSpec((1, 1), lambda i: (0, 0)),
            pl.BlockSpec((D, NOUT), lambda i: (0, 0)),
            pl.BlockSpec((1, NOUT), lambda i: (0, 0)),
            pl.BlockSpec((1, 1, BC), lambda i: (i, 0, 0)),
        ],
        out_specs=pl.BlockSpec((BC, NOUT), lambda i: (i, 0)),
        out_shape=jax.ShapeDtypeStruct((B, NOUT), jnp.float32),
        scratch_shapes=[
            pltpu.VMEM((T, BC, D), jnp.float32),
            pltpu.VMEM((T, BC), jnp.int32),
            pltpu.VMEM((NTAP, T, BC), jnp.float32),
        ],
    )(
        x, W_in, b_in.reshape(1, D), ln_scale.reshape(1, D),
        ln_bias.reshape(1, D), jump_destinations.reshape(1, M),
        W_gate, b_gate.reshape(1, 1), cscale,
        W_out, b_out.reshape(1, NOUT), pointer_init.reshape(GRID, 1, BC),
    )
    aux_loss = jnp.asarray(0.0, x.dtype)
    return logits, aux_loss


# speedup vs baseline: 1.6608x; 1.2213x over previous
import jax
import jax.numpy as jnp
from jax.experimental import pallas as pl
from jax.experimental.pallas import tpu as pltpu

B, T, IN = 1024, 32, 32
M, D, NOUT = 256, 64, 128
K = 2
NTAP = 2 * K + 1
TEMP = 8.0
EPS = 1e-5

BC = 256
GRID = B // BC


def _recurrence_kernel(x_ref, w_in_ref, b_in_ref, ln_s_ref, ln_b_ref,
                       jd_ref, w_gate_ref, b_gate_ref, cscale_ref,
                       w_out_ref, b_out_ref, ptr_ref, out_ref,
                       su_hist, base_hist, w_hist):
    ptr = ptr_ref[0, 0, :]
    hidden = jnp.zeros((BC, D), jnp.float32)
    cscale = cscale_ref[0, 0]

    for t in range(T):
        e = jnp.tanh(
            jnp.dot(x_ref[:, t, :], w_in_ref[...],
                    preferred_element_type=jnp.float32) + b_in_ref[0, :])
        base = jnp.floor(ptr).astype(jnp.int32)
        base = jnp.clip(base, 0, M - 1)
        att = []
        for i in range(NTAP):
            idx_f = jnp.mod(base + (i - K), M).astype(jnp.float32)
            delta = jnp.remainder(idx_f - ptr + M / 2.0, float(M)) - M / 2.0
            att.append(-(delta * delta) / TEMP)
        mx = jnp.maximum(jnp.maximum(jnp.maximum(att[0], att[1]),
                                     jnp.maximum(att[2], att[3])), att[4])
        ex = [jnp.exp(a - mx) for a in att]
        tot = ((ex[0] + ex[1]) + (ex[2] + ex[3])) + ex[4]
        w = [ei / tot for ei in ex]

        if t > 0:
            bh = base_hist[0:t]
            db = jnp.mod(bh - base[None, :], M)
            dd = jnp.where(db > M // 2, db - M, db)
            c = jnp.zeros((t, BC), jnp.float32)
            for d in range(-2 * K, 2 * K + 1):
                lo, hi = max(0, -d), min(NTAP - 1, NTAP - 1 - d)
                corr = jnp.zeros((t, BC), jnp.float32)
                for j in range(lo, hi + 1):
                    corr = corr + w[j + d][None, :] * w_hist[j, 0:t]
                c = jnp.where(dd == d, corr, c)
            context = jnp.sum(c[:, :, None] * su_hist[0:t], axis=0)
            combined = e + cscale * context
        else:
            combined = e + cscale * jnp.zeros((BC, D), jnp.float32)
        su = jnp.tanh(combined + hidden)
        mu = jnp.mean(su, axis=-1, keepdims=True)
        var = jnp.mean((su - mu) ** 2, axis=-1, keepdims=True)
        hidden = (su - mu) / jnp.sqrt(var + EPS) * ln_s_ref[0, :] + ln_b_ref[0, :]

        su_hist[t] = su
        base_hist[t] = base
        for j in range(NTAP):
            w_hist[j, t] = w[j]

        pos = jnp.clip(ptr.astype(jnp.int32), 0, M - 1)
        jt = jnp.sum(
            jnp.where(jax.lax.broadcasted_iota(jnp.int32, (BC, M), 1) == pos[:, None],
                      jd_ref[0, :][None, :], 0.0), axis=1)
        jl = (jnp.dot(su, w_gate_ref[...],
                      preferred_element_type=jnp.float32)[:, 0] + b_gate_ref[0, 0])
        walk = jnp.remainder(ptr + 1.0, float(M))
        ptr = jnp.where(jl > 0.0, jt, walk)

    out_ref[...] = (jnp.dot(hidden, w_out_ref[...],
                            preferred_element_type=jnp.float32) + b_out_ref[0, :])


def kernel(x, W_in, b_in, ln_scale, ln_bias, jump_destinations, W_gate,
           b_gate, context_strength, W_out, b_out, pointer_init, holonomy_init):
    cscale = jax.nn.sigmoid(context_strength).reshape(1, 1)
    logits = pl.pallas_call(
        _recurrence_kernel,
        grid=(GRID,),
        in_specs=[
            pl.BlockSpec((BC, T, IN), lambda i: (i, 0, 0)),
            pl.BlockSpec((IN, D), lambda i: (0, 0)),
            pl.BlockSpec((1, D), lambda i: (0, 0)),
            pl.BlockSpec((1, D), lambda i: (0, 0)),
            pl.BlockSpec((1, D), lambda i: (0, 0)),
            pl.BlockSpec((1, M), lambda i: (0, 0)),
            pl.BlockSpec((D, 1), lambda i: (0, 0)),
            pl.BlockSpec((1, 1), lambda i: (0, 0)),
            pl.BlockSpec((1, 1), lambda i: (0, 0)),
            pl.BlockSpec((D, NOUT), lambda i: (0, 0)),
            pl.BlockSpec((1, NOUT), lambda i: (0, 0)),
            pl.BlockSpec((1, 1, BC), lambda i: (i, 0, 0)),
        ],
        out_specs=pl.BlockSpec((BC, NOUT), lambda i: (i, 0)),
        out_shape=jax.ShapeDtypeStruct((B, NOUT), jnp.float32),
        scratch_shapes=[
            pltpu.VMEM((T, BC, D), jnp.float32),
            pltpu.VMEM((T, BC), jnp.int32),
            pltpu.VMEM((NTAP, T, BC), jnp.float32),
        ],
    )(
        x, W_in, b_in.reshape(1, D), ln_scale.reshape(1, D),
        ln_bias.reshape(1, D), jump_destinations.reshape(1, M),
        W_gate, b_gate.reshape(1, 1), cscale,
        W_out, b_out.reshape(1, NOUT), pointer_init.reshape(GRID, 1, BC),
    )
    aux_loss = jnp.asarray(0.0, x.dtype)
    return logits, aux_loss


# speedup vs baseline: 34.9344x; 21.0351x over previous
import jax
import jax.numpy as jnp
from jax.experimental import pallas as pl
from jax.experimental.pallas import tpu as pltpu

B, T, IN = 1024, 32, 32
M, D, NOUT = 256, 64, 128
K = 2
NTAP = 2 * K + 1
TEMP = 8.0
EPS = 1e-5

BC = 256
GRID = B // BC


def _recurrence_kernel(xt_ref, w_in_t_ref, b_in_ref, ln_s_ref, ln_b_ref,
                       jd_ref, w_gate_t_ref, b_gate_ref, cscale_ref,
                       w_out_t_ref, b_out_ref, ptr_ref, out_ref,
                       su_hist, base_hist, w_hist, emb_ref):
    ptr = ptr_ref[0, 0, :]
    hidden = jnp.zeros((D, BC), jnp.float32)
    cscale = cscale_ref[0, 0]

    emb_ref[...] = jnp.tanh(
        jnp.dot(w_in_t_ref[...], xt_ref[...].reshape(IN, T * BC),
                preferred_element_type=jnp.float32) + b_in_ref[...])

    for t in range(T):
        e = emb_ref[:, t * BC:(t + 1) * BC]
        base = jnp.floor(ptr).astype(jnp.int32)
        base = jnp.clip(base, 0, M - 1)
        att = []
        for i in range(NTAP):
            idx_f = jnp.mod(base + (i - K), M).astype(jnp.float32)
            delta = jnp.remainder(idx_f - ptr + M / 2.0, float(M)) - M / 2.0
            att.append(-(delta * delta) / TEMP)
        mx = jnp.maximum(jnp.maximum(jnp.maximum(att[0], att[1]),
                                     jnp.maximum(att[2], att[3])), att[4])
        ex = [jnp.exp(a - mx) for a in att]
        tot = ((ex[0] + ex[1]) + (ex[2] + ex[3])) + ex[4]
        w = [ei / tot for ei in ex]

        if t > 0:
            bh = base_hist[0:t]
            db = jnp.mod(bh - base[None, :], M)
            dd = jnp.where(db > M // 2, db - M, db)
            c = jnp.zeros((t, BC), jnp.float32)
            for d in range(-2 * K, 2 * K + 1):
                lo, hi = max(0, -d), min(NTAP - 1, NTAP - 1 - d)
                corr = jnp.zeros((t, BC), jnp.float32)
                for j in range(lo, hi + 1):
                    corr = corr + w[j + d][None, :] * w_hist[j, 0:t]
                c = jnp.where(dd == d, corr, c)
            context = jnp.sum(c[:, None, :] * su_hist[0:t], axis=0)
            combined = e + cscale * context
        else:
            combined = e + cscale * jnp.zeros((D, BC), jnp.float32)

        su = jnp.tanh(combined + hidden)
        mu = jnp.mean(su, axis=0, keepdims=True)
        var = jnp.mean((su - mu) ** 2, axis=0, keepdims=True)
        hidden = (su - mu) / jnp.sqrt(var + EPS) * ln_s_ref[...] + ln_b_ref[...]

        su_hist[t] = su
        base_hist[t] = base
        for j in range(NTAP):
            w_hist[j, t] = w[j]

        pos = jnp.clip(ptr.astype(jnp.int32), 0, M - 1)
        jt = jnp.sum(
            jnp.where(jax.lax.broadcasted_iota(jnp.int32, (BC, M), 1) == pos[:, None],
                      jd_ref[0, :][None, :], 0.0), axis=1)
        jl = (jnp.dot(w_gate_t_ref[...], su,
                      preferred_element_type=jnp.float32)[0, :] + b_gate_ref[0, 0])
        walk = jnp.remainder(ptr + 1.0, float(M))
        ptr = jnp.where(jl > 0.0, jt, walk)

    out_ref[...] = (jnp.dot(w_out_t_ref[...], hidden,
                            preferred_element_type=jnp.float32) + b_out_ref[...])


def kernel(x, W_in, b_in, ln_scale, ln_bias, jump_destinations, W_gate,
           b_gate, context_strength, W_out, b_out, pointer_init, holonomy_init):
    cscale = jax.nn.sigmoid(context_strength).reshape(1, 1)
    xt = x.transpose(2, 1, 0)
    logits_t = pl.pallas_call(
        _recurrence_kernel,
        grid=(GRID,),
        in_specs=[
            pl.BlockSpec((IN, T, BC), lambda i: (0, 0, i)),
            pl.BlockSpec((D, IN), lambda i: (0, 0)),
            pl.BlockSpec((D, 1), lambda i: (0, 0)),
            pl.BlockSpec((D, 1), lambda i: (0, 0)),
            pl.BlockSpec((D, 1), lambda i: (0, 0)),
            pl.BlockSpec((1, M), lambda i: (0, 0)),
            pl.BlockSpec((1, D), lambda i: (0, 0)),
            pl.BlockSpec((1, 1), lambda i: (0, 0)),
            pl.BlockSpec((1, 1), lambda i: (0, 0)),
            pl.BlockSpec((NOUT, D), lambda i: (0, 0)),
            pl.BlockSpec((NOUT, 1), lambda i: (0, 0)),
            pl.BlockSpec((1, 1, BC), lambda i: (i, 0, 0)),
        ],
        out_specs=pl.BlockSpec((NOUT, BC), lambda i: (0, i)),
        out_shape=jax.ShapeDtypeStruct((NOUT, B), jnp.float32),
        scratch_shapes=[
            pltpu.VMEM((T, D, BC), jnp.float32),
            pltpu.VMEM((T, BC), jnp.int32),
            pltpu.VMEM((NTAP, T, BC), jnp.float32),
            pltpu.VMEM((D, T * BC), jnp.float32),
        ],
    )(
        xt, W_in.T, b_in.reshape(D, 1), ln_scale.reshape(D, 1),
        ln_bias.reshape(D, 1), jump_destinations.reshape(1, M),
        W_gate.reshape(1, D), b_gate.reshape(1, 1), cscale,
        W_out.T, b_out.reshape(NOUT, 1), pointer_init.reshape(GRID, 1, BC),
    )
    logits = logits_t.T
    aux_loss = jnp.asarray(0.0, x.dtype)
    return logits, aux_loss


# speedup vs baseline: 36.1639x; 1.0352x over previous
import jax
import jax.numpy as jnp
from jax.experimental import pallas as pl
from jax.experimental.pallas import tpu as pltpu

B, T, IN = 1024, 32, 32
M, D, NOUT = 256, 64, 128
K = 2
NTAP = 2 * K + 1
TEMP = 8.0
EPS = 1e-5

BC = 1024
GRID = B // BC


def _recurrence_kernel(xt_ref, w_in_t_ref, b_in_ref, ln_s_ref, ln_b_ref,
                       jd_ref, w_gate_t_ref, b_gate_ref, cscale_ref,
                       w_out_t_ref, b_out_ref, ptr_ref, out_ref,
                       su_hist, base_hist, w_hist, emb_ref):
    ptr = ptr_ref[0, 0, :]
    hidden = jnp.zeros((D, BC), jnp.float32)
    cscale = cscale_ref[0, 0]

    emb_ref[...] = jnp.tanh(
        jnp.dot(w_in_t_ref[...], xt_ref[...].reshape(IN, T * BC),
                preferred_element_type=jnp.float32) + b_in_ref[...])

    for t in range(T):
        e = emb_ref[:, t * BC:(t + 1) * BC]
        base = jnp.floor(ptr).astype(jnp.int32)
        base = jnp.clip(base, 0, M - 1)
        att = []
        for i in range(NTAP):
            idx_f = jnp.mod(base + (i - K), M).astype(jnp.float32)
            delta = jnp.remainder(idx_f - ptr + M / 2.0, float(M)) - M / 2.0
            att.append(-(delta * delta) / TEMP)
        mx = jnp.maximum(jnp.maximum(jnp.maximum(att[0], att[1]),
                                     jnp.maximum(att[2], att[3])), att[4])
        ex = [jnp.exp(a - mx) for a in att]
        tot = ((ex[0] + ex[1]) + (ex[2] + ex[3])) + ex[4]
        w = [ei / tot for ei in ex]

        if t > 0:
            bh = base_hist[0:t]
            db = jnp.mod(bh - base[None, :], M)
            dd = jnp.where(db > M // 2, db - M, db)
            c = jnp.zeros((t, BC), jnp.float32)
            for d in range(-2 * K, 2 * K + 1):
                lo, hi = max(0, -d), min(NTAP - 1, NTAP - 1 - d)
                corr = jnp.zeros((t, BC), jnp.float32)
                for j in range(lo, hi + 1):
                    corr = corr + w[j + d][None, :] * w_hist[j, 0:t]
                c = jnp.where(dd == d, corr, c)
            context = jnp.sum(c[:, None, :] * su_hist[0:t], axis=0)
            combined = e + cscale * context
        else:
            combined = e + cscale * jnp.zeros((D, BC), jnp.float32)

        su = jnp.tanh(combined + hidden)
        mu = jnp.mean(su, axis=0, keepdims=True)
        var = jnp.mean((su - mu) ** 2, axis=0, keepdims=True)
        hidden = (su - mu) / jnp.sqrt(var + EPS) * ln_s_ref[...] + ln_b_ref[...]

        su_hist[t] = su
        base_hist[t] = base
        for j in range(NTAP):
            w_hist[j, t] = w[j]

        pos = jnp.clip(ptr.astype(jnp.int32), 0, M - 1)
        jt = jnp.sum(
            jnp.where(jax.lax.broadcasted_iota(jnp.int32, (BC, M), 1) == pos[:, None],
                      jd_ref[0, :][None, :], 0.0), axis=1)
        jl = (jnp.dot(w_gate_t_ref[...], su,
                      preferred_element_type=jnp.float32)[0, :] + b_gate_ref[0, 0])
        walk = jnp.remainder(ptr + 1.0, float(M))
        ptr = jnp.where(jl > 0.0, jt, walk)

    out_ref[...] = (jnp.dot(w_out_t_ref[...], hidden,
                            preferred_element_type=jnp.float32) + b_out_ref[...])


def kernel(x, W_in, b_in, ln_scale, ln_bias, jump_destinations, W_gate,
           b_gate, context_strength, W_out, b_out, pointer_init, holonomy_init):
    cscale = jax.nn.sigmoid(context_strength).reshape(1, 1)
    xt = x.transpose(2, 1, 0)
    logits_t = pl.pallas_call(
        _recurrence_kernel,
        grid=(GRID,),
        in_specs=[
            pl.BlockSpec((IN, T, BC), lambda i: (0, 0, i)),
            pl.BlockSpec((D, IN), lambda i: (0, 0)),
            pl.BlockSpec((D, 1), lambda i: (0, 0)),
            pl.BlockSpec((D, 1), lambda i: (0, 0)),
            pl.BlockSpec((D, 1), lambda i: (0, 0)),
            pl.BlockSpec((1, M), lambda i: (0, 0)),
            pl.BlockSpec((1, D), lambda i: (0, 0)),
            pl.BlockSpec((1, 1), lambda i: (0, 0)),
            pl.BlockSpec((1, 1), lambda i: (0, 0)),
            pl.BlockSpec((NOUT, D), lambda i: (0, 0)),
            pl.BlockSpec((NOUT, 1), lambda i: (0, 0)),
            pl.BlockSpec((1, 1, BC), lambda i: (i, 0, 0)),
        ],
        out_specs=pl.BlockSpec((NOUT, BC), lambda i: (0, i)),
        out_shape=jax.ShapeDtypeStruct((NOUT, B), jnp.float32),
        scratch_shapes=[
            pltpu.VMEM((T, D, BC), jnp.float32),
            pltpu.VMEM((T, BC), jnp.int32),
            pltpu.VMEM((NTAP, T, BC), jnp.float32),
            pltpu.VMEM((D, T * BC), jnp.float32),
        ],
    )(
        xt, W_in.T, b_in.reshape(D, 1), ln_scale.reshape(D, 1),
        ln_bias.reshape(D, 1), jump_destinations.reshape(1, M),
        W_gate.reshape(1, D), b_gate.reshape(1, 1), cscale,
        W_out.T, b_out.reshape(NOUT, 1), pointer_init.reshape(GRID, 1, BC),
    )
    logits = logits_t.T
    aux_loss = jnp.asarray(0.0, x.dtype)
    return logits, aux_loss
